# Initial kernel scaffold; baseline (speedup 1.0000x reference)
import functools

import jax
import jax.numpy as jnp
from jax import lax
from jax.experimental import pallas as pl
from jax.experimental.pallas import tpu as pltpu
from jax.experimental.pallas import tpu_sc as plsc

N_SEG = 10000
N_ELEM = 320000
D = 128

NC = 2
NS = 16
ROWS_PER_TILE = N_ELEM // (NC * NS)
CHUNK = 128
N_FULL = ROWS_PER_TILE // CHUNK
TAIL = ROWS_PER_TILE - N_FULL * CHUNK
SLAB = N_SEG // NS
ZROWS = 125


def _sc_body(v_hbm, b_hbm, p_hbm, table, vbuf, idx_v, idx_t):
    c = lax.axis_index("c")
    s = lax.axis_index("s")
    wid = c * NS + s
    row0 = wid * ROWS_PER_TILE

    def zero_row(r, _):
        for k in range(D // 16):
            vbuf[r, pl.ds(k * 16, 16)] = jnp.zeros((16,), jnp.float32)
        return 0

    lax.fori_loop(0, CHUNK, zero_row, 0)

    for j in range(SLAB // ZROWS):
        pltpu.sync_copy(vbuf.at[pl.ds(0, ZROWS)],
                        table.at[pl.ds(s * SLAB + j * ZROWS, ZROWS)])
    plsc.subcore_barrier()

    def chunk_body(i, _):
        base = pl.multiple_of(row0 + i * CHUNK, 8)
        pltpu.sync_copy(b_hbm.at[pl.ds(base, CHUNK)], idx_v)
        pltpu.sync_copy(v_hbm.at[pl.ds(base, CHUNK)], vbuf)
        pltpu.sync_copy(vbuf, table.at[idx_v], add=True)
        return 0

    lax.fori_loop(0, N_FULL, chunk_body, 0)

    base_t = pl.multiple_of(row0 + N_FULL * CHUNK, 8)
    pltpu.sync_copy(b_hbm.at[pl.ds(base_t, TAIL)], idx_t)
    pltpu.sync_copy(v_hbm.at[pl.ds(base_t, TAIL)], vbuf.at[pl.ds(0, TAIL)])
    pltpu.sync_copy(vbuf.at[pl.ds(0, TAIL)], table.at[idx_t], add=True)

    plsc.subcore_barrier()

    for j in range(SLAB // ZROWS):
        r = s * SLAB + j * ZROWS
        pltpu.sync_copy(table.at[pl.ds(r, ZROWS)],
                        vbuf.at[pl.ds(0, ZROWS)])
        pltpu.sync_copy(vbuf.at[pl.ds(0, ZROWS)],
                        p_hbm.at[c, pl.ds(r, ZROWS)])


_sc_scatter = functools.partial(
    pl.kernel,
    out_type=jax.ShapeDtypeStruct((NC, N_SEG, D), jnp.float32),
    mesh=plsc.VectorSubcoreMesh(core_axis_name="c", subcore_axis_name="s"),
    scratch_types=[
        pltpu.VMEM_SHARED((N_SEG, D), jnp.float32),
        pltpu.VMEM((CHUNK, D), jnp.float32),
        pltpu.VMEM((CHUNK,), jnp.int32),
        pltpu.VMEM((TAIL,), jnp.int32),
    ],
)(_sc_body)


def _combine_body(u_ref, p0_ref, p1_ref, o_ref):
    o_ref[...] = u_ref[...] + p0_ref[...] + p1_ref[...]


def _combine(u, p0, p1):
    blk = 1250
    spec = pl.BlockSpec((blk, D), lambda i: (i, 0))
    return pl.pallas_call(
        _combine_body,
        grid=(N_SEG // blk,),
        in_specs=[spec, spec, spec],
        out_specs=spec,
        out_shape=jax.ShapeDtypeStruct((N_SEG, D), jnp.float32),
    )(u, p0, p1)


@jax.jit
def kernel(u, v, batch):
    b32 = batch.astype(jnp.int32)
    p = _sc_scatter(v, b32)
    return _combine(u, p[0], p[1])


# speedup vs baseline: 4.3973x; 4.3973x over previous
import functools

import jax
import jax.numpy as jnp
from jax import lax
from jax.experimental import pallas as pl
from jax.experimental.pallas import tpu as pltpu
from jax.experimental.pallas import tpu_sc as plsc

N_SEG = 10000
N_ELEM = 320000
D = 128

NC = 2
NS = 16
ROWS_PER_TILE = N_ELEM // (NC * NS)
CHUNK = 128
N_FULL = ROWS_PER_TILE // CHUNK
TAIL = ROWS_PER_TILE - N_FULL * CHUNK
SLAB = 624
SLAB_CHUNKS = ((0, 128), (128, 128), (256, 128), (384, 128), (512, 112))
REM_START = NS * SLAB
REM = N_SEG - REM_START


def _sc_body(v_hbm, b_hbm, p_hbm, table, vbuf, idx_v, idx_t):
    c = lax.axis_index("c")
    s = lax.axis_index("s")
    wid = c * NS + s
    row0 = wid * ROWS_PER_TILE

    def zero_row(r, _):
        for k in range(D // 16):
            vbuf[r, pl.ds(k * 16, 16)] = jnp.zeros((16,), jnp.float32)
        return 0

    lax.fori_loop(0, CHUNK, zero_row, 0)

    for off, sz in SLAB_CHUNKS:
        pltpu.sync_copy(vbuf.at[pl.ds(0, sz)],
                        table.at[pl.ds(s * SLAB + off, sz)])

    @pl.when(s == NS - 1)
    def _zero_rem():
        pltpu.sync_copy(vbuf.at[pl.ds(0, REM)],
                        table.at[pl.ds(REM_START, REM)])

    plsc.subcore_barrier()

    def chunk_body(i, _):
        base = pl.multiple_of(row0 + i * CHUNK, 8)
        pltpu.sync_copy(b_hbm.at[pl.ds(base, CHUNK)], idx_v)
        pltpu.sync_copy(v_hbm.at[pl.ds(base, CHUNK)], vbuf)
        pltpu.sync_copy(vbuf, table.at[idx_v], add=True)
        return 0

    lax.fori_loop(0, N_FULL, chunk_body, 0)

    base_t = pl.multiple_of(row0 + N_FULL * CHUNK, 8)
    pltpu.sync_copy(b_hbm.at[pl.ds(base_t, TAIL)], idx_t)
    pltpu.sync_copy(v_hbm.at[pl.ds(base_t, TAIL)], vbuf.at[pl.ds(0, TAIL)])
    pltpu.sync_copy(vbuf.at[pl.ds(0, TAIL)], table.at[idx_t], add=True)

    plsc.subcore_barrier()

    for off, sz in SLAB_CHUNKS:
        r = s * SLAB + off
        pltpu.sync_copy(table.at[pl.ds(r, sz)], vbuf.at[pl.ds(0, sz)])
        pltpu.sync_copy(vbuf.at[pl.ds(0, sz)], p_hbm.at[c, pl.ds(r, sz)])

    @pl.when(s == NS - 1)
    def _write_rem():
        pltpu.sync_copy(table.at[pl.ds(REM_START, REM)],
                        vbuf.at[pl.ds(0, REM)])
        pltpu.sync_copy(vbuf.at[pl.ds(0, REM)],
                        p_hbm.at[c, pl.ds(REM_START, REM)])


_sc_scatter = functools.partial(
    pl.kernel,
    out_type=jax.ShapeDtypeStruct((NC, N_SEG, D), jnp.float32),
    mesh=plsc.VectorSubcoreMesh(core_axis_name="c", subcore_axis_name="s"),
    scratch_types=[
        pltpu.VMEM_SHARED((N_SEG, D), jnp.float32),
        pltpu.VMEM((CHUNK, D), jnp.float32),
        pltpu.VMEM((CHUNK,), jnp.int32),
        pltpu.VMEM((TAIL,), jnp.int32),
    ],
)(_sc_body)


def _combine_body(u_ref, p0_ref, p1_ref, o_ref):
    o_ref[...] = u_ref[...] + p0_ref[...] + p1_ref[...]


def _combine(u, p0, p1):
    blk = 1000
    spec = pl.BlockSpec((blk, D), lambda i: (i, 0))
    return pl.pallas_call(
        _combine_body,
        grid=(N_SEG // blk,),
        in_specs=[spec, spec, spec],
        out_specs=spec,
        out_shape=jax.ShapeDtypeStruct((N_SEG, D), jnp.float32),
    )(u, p0, p1)


@jax.jit
def kernel(u, v, batch):
    b32 = batch.astype(jnp.int32)
    p = _sc_scatter(v, b32)
    return _combine(u, p[0], p[1])
